# baseline (speedup 1.0000x reference)
import jax
import jax.numpy as jnp
from jax.experimental import pallas as pl
from jax.experimental.pallas import tpu as pltpu

G = 16
K = 3
S = 2
C = 3
IMG = 224
H_G = 128
STD = 0.17
HIDDEN = 256
NCLS = 1000


def _select_pool_rows(d0, f):
    g = jax.lax.broadcasted_iota(jnp.int32, (G, IMG), 0)
    u = jax.lax.broadcasted_iota(jnp.int32, (G, IMG), 1)
    q = u - d0 - g * f
    sel = jnp.logical_and(q >= 0, q < f)
    return jnp.where(sel, jnp.float32(1.0 / f), jnp.float32(0.0))


def _select_pool_cols(d1, f):
    j = jax.lax.broadcasted_iota(jnp.int32, (IMG * C, G * C), 0)
    t = jax.lax.broadcasted_iota(jnp.int32, (IMG * C, G * C), 1)
    w = j // C
    cj = j - w * C
    g2 = t // C
    ct = t - g2 * C
    q = w - d1 - g2 * f
    sel = jnp.logical_and(jnp.logical_and(q >= 0, q < f), cj == ct)
    return jnp.where(sel, jnp.float32(1.0 / f), jnp.float32(0.0))


def _chain_gcn_matrix():
    n = 7 + 1
    r = jax.lax.broadcasted_iota(jnp.int32, (n, n), 0)
    c = jax.lax.broadcasted_iota(jnp.int32, (n, n), 1)
    inv_sqrt2 = 1.0 / jnp.sqrt(jnp.float32(2.0))
    diag = jnp.where(r == c, jnp.where(r == 0, 1.0, 0.5), 0.0)
    sub = jnp.where(r == c + 1, jnp.where(r == 1, inv_sqrt2, 0.5), 0.0)
    return (diag + sub).astype(jnp.float32)


_B1, _B2, _BL1, _W2, _WL1, _WL2, _WB = 0, 8, 16, 24, 32, 288, 416
_B3, _B4, _BG1, _BG2, _W3, _W4, _WG1, _WG2 = 0, 8, 16, 24, 32, 160, 288, 544
_BC, _WC = 0, 8


def _fwd_kernel(s_ref, x_ref, snaps_ref, noise_ref, w1p_ref,
                p128_ref, p256_ref, p1000_ref,
                out_h, out_l, out_b, out_p, out_pi):
    f32 = jnp.float32

    ly = s_ref[0, 0]
    lx = s_ref[0, 1]

    def start(coord, size):
        center = 0.5 * ((coord + 1.0) * IMG)
        n = center.astype(jnp.int32)
        frac = center - n.astype(f32)
        odd = jnp.bitwise_and(n, 1)
        rnd = n + jnp.where(frac > 0.5, 1, jnp.where(frac == 0.5, odd, 0))
        st = rnd - size // 2 + size
        return jnp.clip(st, 0, IMG + size) - size

    r_blk = jax.lax.broadcasted_iota(jnp.int32, (G, G * H_G), 0)
    c_blk = jax.lax.broadcasted_iota(jnp.int32, (G, G * H_G), 1)
    diag_mask = (c_blk // H_G) == r_blk
    j_id = jax.lax.broadcasted_iota(jnp.int32, (G * H_G, H_G), 0)
    o_id = jax.lax.broadcasted_iota(jnp.int32, (G * H_G, H_G), 1)
    block_id = jnp.where(j_id % H_G == o_id, 1.0, 0.0).astype(f32)

    x2 = x_ref[...]
    g1v = p128_ref[_B1:_B1 + 1, :]
    for i in range(K):
        size = G * (S ** i)
        f = size // G
        d0 = start(ly, size)
        d1 = start(lx, size)
        pr = _select_pool_rows(d0, f)
        pct = _select_pool_cols(d1, f)
        pooled = jax.lax.dot(jax.lax.dot(pr, x2), pct)
        q = jax.lax.dot(pooled, w1p_ref[i])
        s = jnp.sum(jnp.where(diag_mask, q, 0.0), axis=0, keepdims=True)
        g1v = g1v + jax.lax.dot(s, block_id)
    g1v = jnp.maximum(g1v, 0.0)

    l1 = jnp.maximum(p128_ref[_W2:_W2 + 1, :] * ly
                     + p128_ref[_W2 + 1:_W2 + 2, :] * lx
                     + p128_ref[_B2:_B2 + 1, :], 0.0)

    g_t = jnp.maximum(
        (jax.lax.dot(g1v, p256_ref[_W3:_W3 + H_G, :])
         + p256_ref[_B3:_B3 + 1, :])
        + (jax.lax.dot(l1, p256_ref[_W4:_W4 + H_G, :])
           + p256_ref[_B4:_B4 + 1, :]), 0.0)

    nf = jnp.concatenate([snaps_ref[...], g_t], axis=0)
    A = _chain_gcn_matrix()
    h1 = jnp.maximum(
        jax.lax.dot(A, jax.lax.dot(nf, p256_ref[_WG1:_WG1 + HIDDEN, :]))
        + p256_ref[_BG1:_BG1 + 1, :], 0.0)
    out2 = (jax.lax.dot(A, jax.lax.dot(h1, p256_ref[_WG2:_WG2 + HIDDEN, :]))
            + p256_ref[_BG2:_BG2 + 1, :])
    h_t = jnp.mean(out2, axis=0, keepdims=True)
    out_h[...] = h_t

    feat = jnp.maximum(jax.lax.dot(h_t, p128_ref[_WL1:_WL1 + HIDDEN, :])
                       + p128_ref[_BL1:_BL1 + 1, :], 0.0)
    mu_full = jax.lax.dot(feat, p128_ref[_WL2:_WL2 + H_G, :])
    z11 = jnp.zeros((1, 1), f32)
    bl2v = jnp.concatenate([z11 + s_ref[0, 2], z11 + s_ref[0, 3]], axis=1)
    mu = jnp.tanh(mu_full[:, 0:2] + bl2v)
    l_pre = mu + STD * noise_ref[...]
    out_l[...] = jnp.clip(l_pre, -1.0, 1.0)
    z = (l_pre - mu) / STD
    terms = -0.5 * z * z - jnp.log(f32(STD)) - 0.5 * jnp.log(2.0 * f32(jnp.pi))
    out_pi[...] = jnp.sum(terms, axis=1, keepdims=True)

    b_full = jax.lax.dot(h_t, p128_ref[_WB:_WB + HIDDEN, :])
    out_b[...] = b_full[:, 0:1] + s_ref[0, 4]

    logits = (jax.lax.dot(h_t, p1000_ref[_WC:_WC + HIDDEN, :])
              + p1000_ref[_BC:_BC + 1, :])
    m = jnp.max(logits, axis=1, keepdims=True)
    sh = logits - m
    out_p[...] = sh - jnp.log(jnp.sum(jnp.exp(sh), axis=1, keepdims=True))


def kernel(x, l_t_prev, h_t_prev, snaps_prev, noise, params):
    del h_t_prev
    p = params
    f32 = jnp.float32

    xr = x[0].reshape(IMG, IMG * C).astype(f32)

    w1p = (p['W1'].reshape(K, G, G * C, H_G)
           .transpose(0, 2, 1, 3)
           .reshape(K, G * C, G * H_G))

    def row(v):
        return v.reshape(1, -1).astype(f32)

    def padrows(a, rows):
        return jnp.concatenate(
            [a, jnp.zeros((rows - a.shape[0], a.shape[1]), f32)], axis=0)

    def padcols(a, cols):
        return jnp.concatenate(
            [a, jnp.zeros((a.shape[0], cols - a.shape[1]), f32)], axis=1)

    p128 = jnp.concatenate([
        padrows(row(p['b1']), 8), padrows(row(p['b2']), 8),
        padrows(row(p['bl1']), 8), padrows(p['W2'].astype(f32), 8),
        p['Wl1'].astype(f32),
        padcols(p['Wl2'].astype(f32), H_G),
        padcols(p['Wb'].astype(f32), H_G),
    ], axis=0)
    p256 = jnp.concatenate([
        padrows(row(p['b3']), 8), padrows(row(p['b4']), 8),
        padrows(padcols(row(p['bg1']), HIDDEN), 8), padrows(row(p['bg2']), 8),
        p['W3'].astype(f32), p['W4'].astype(f32),
        padcols(p['Wg1'].astype(f32), HIDDEN),
        padrows(p['Wg2'].astype(f32), HIDDEN),
    ], axis=0)
    p1000 = jnp.concatenate([
        padrows(row(p['bc']), 8), p['Wc'].astype(f32),
    ], axis=0)

    svec = jnp.concatenate([
        l_t_prev[0:1].astype(f32).reshape(1, 2),
        p['bl2'].astype(f32).reshape(1, 2),
        p['bb'].astype(f32).reshape(1, 1),
        jnp.zeros((1, 3), f32),
    ], axis=1)

    out_shapes = (
        jax.ShapeDtypeStruct((1, HIDDEN), f32),
        jax.ShapeDtypeStruct((1, 2), f32),
        jax.ShapeDtypeStruct((1, 1), f32),
        jax.ShapeDtypeStruct((1, NCLS), f32),
        jax.ShapeDtypeStruct((1, 1), f32),
    )
    in_specs = ([pl.BlockSpec(memory_space=pltpu.SMEM)] +
                [pl.BlockSpec(memory_space=pltpu.VMEM) for _ in range(7)])

    h_t, l_t, b_t, log_probas, log_pi = pl.pallas_call(
        _fwd_kernel,
        out_shape=out_shapes,
        in_specs=in_specs,
        out_specs=tuple(pl.BlockSpec(memory_space=pltpu.VMEM)
                        for _ in range(5)),
    )(svec, xr, snaps_prev.astype(f32), noise.astype(f32), w1p,
      p128, p256, p1000)

    return (h_t, l_t, b_t.reshape(()), log_probas, log_pi.reshape((1,)))
